# baseline (speedup 1.0000x reference)
import functools

import jax
import jax.numpy as jnp
from jax import lax
from jax.experimental import pallas as pl
from jax.experimental.pallas import tpu as pltpu
from jax.experimental.pallas import tpu_sc as plsc

NC, NS, LANES = 2, 16, 16
NW = NC * NS
CH = 16


def kernel(input_seq, pe):
    B, S, D = input_seq.shape
    R = B * S
    x2 = input_seq.reshape(R, D)
    rpw = R // NW
    nit = rpw // CH
    vregs_per_chunk = CH * D // LANES
    cols = D // LANES
    mesh = plsc.VectorSubcoreMesh(core_axis_name="c", subcore_axis_name="s")

    @functools.partial(
        pl.kernel,
        mesh=mesh,
        out_type=jax.ShapeDtypeStruct((R, D), jnp.float32),
        scratch_types=[
            pltpu.VMEM((CH, D), jnp.float32),
            pltpu.VMEM((CH, D), jnp.float32),
        ],
    )
    def k(x_hbm, pe_hbm, o_hbm, xv, pv):
        wid = lax.axis_index("s") * NC + lax.axis_index("c")
        base = wid * rpw
        pbase = lax.rem(base, S)

        def body(i, carry):
            r = base + i * CH
            pr = pbase + i * CH
            pltpu.sync_copy(x_hbm.at[pl.ds(r, CH)], xv)
            pltpu.sync_copy(pe_hbm.at[pl.ds(pr, CH)], pv)

            def inner(j, c2):
                row = j // cols
                col = lax.rem(j, cols) * LANES
                xv[row, pl.ds(col, LANES)] = (
                    xv[row, pl.ds(col, LANES)] + pv[row, pl.ds(col, LANES)]
                )
                return c2

            lax.fori_loop(0, vregs_per_chunk, inner, 0)
            pltpu.sync_copy(xv, o_hbm.at[pl.ds(r, CH)])
            return carry

        lax.fori_loop(0, nit, body, 0)

    out2 = k(x2, pe)
    return out2.reshape(B, S, D)
